# Initial kernel scaffold; baseline (speedup 1.0000x reference)
import jax
import jax.numpy as jnp
from jax.experimental import pallas as pl


def kernel(X, W1, Wout, homo, H1_V, H1_E, H2_V, H2_E):
    raise NotImplementedError("write your pallas kernel here")


# speedup vs baseline: 1.2587x; 1.2587x over previous
import functools

import jax
import jax.numpy as jnp
from jax.experimental import pallas as pl

N = 10000
NE = 5000
NFEAT = 128
NHID = 128
NCLASS = 64

ROWS_PER_BLK = 1000


def _mm_kernel(x_ref, w_ref, o_ref):
    o_ref[...] = jnp.dot(x_ref[...], w_ref[...],
                         preferred_element_type=jnp.float32)


def _matmul(x, w):
    m, k = x.shape
    _, n = w.shape
    grid = (m // ROWS_PER_BLK,)
    return pl.pallas_call(
        _mm_kernel,
        grid=grid,
        in_specs=[pl.BlockSpec((ROWS_PER_BLK, k), lambda i: (i, 0)),
                  pl.BlockSpec((k, n), lambda i: (0, 0))],
        out_specs=pl.BlockSpec((ROWS_PER_BLK, n), lambda i: (i, 0)),
        out_shape=jax.ShapeDtypeStruct((m, n), jnp.float32),
    )(x, w)


def _conv(Y, vertex, edges, homo, inv_cnt_e, hv):
    Yve = jnp.take(Y, vertex, axis=0)
    Se = jax.ops.segment_sum(Yve, edges, num_segments=NE)
    Xe = Se * (homo * inv_cnt_e)[:, None]
    Xev = jnp.take(Xe, edges, axis=0)
    Sv = jax.ops.segment_sum(Xev, vertex, num_segments=N)
    return Y + Sv * hv[:, None]


def _view_scalars(vertex, edges, homo):
    ones = jnp.ones((vertex.shape[0],), jnp.float32)
    cnt_e = jax.ops.segment_sum(ones, edges, num_segments=NE)
    cnt_v = jax.ops.segment_sum(ones, vertex, num_segments=N)
    he = jnp.take(homo, edges, axis=0)
    att_sum = jax.ops.segment_sum(he, vertex, num_segments=N)
    inv_cnt_e = 1.0 / jnp.clip(cnt_e, 1.0, None)
    hv = 1.0 / (jnp.maximum(att_sum, 1e-30) * jnp.clip(cnt_v, 1.0, None))
    return inv_cnt_e, hv


def kernel(X, W1, Wout, homo, H1_V, H1_E, H2_V, H2_E):
    Y1 = _matmul(X, W1)
    outs = []
    for (V, E) in ((H1_V, H1_E), (H2_V, H2_E)):
        inv_cnt_e, hv = _view_scalars(V, E, homo)
        Xl = jax.nn.relu(_conv(Y1, V, E, homo, inv_cnt_e, hv))
        Y2 = _matmul(Xl, Wout)
        Xo = _conv(Y2, V, E, homo, inv_cnt_e, hv)
        outs.append((Xl, Xo))
    (Z1, X1o), (Z2, X2o) = outs
    return (Z1, Z2, X1o, X2o)


# speedup vs baseline: 10.8060x; 8.5852x over previous
import functools

import jax
import jax.numpy as jnp
from jax import lax
from jax.experimental import pallas as pl
from jax.experimental.pallas import tpu as pltpu
from jax.experimental.pallas import tpu_sc as plsc

N = 10000
NE = 5000
INC = 320000
NFEAT = 128
NHID = 128
NCLASS = 64

D1 = NHID + 16
D2 = NCLASS
SPE = 5120
NP = 10240
NSUB = 16
PER_SUB = INC // NSUB
CH = 80
NCHUNK = PER_SUB // CH
IB = 10
NB = NCHUNK // IB


def _make_sc_stage(D, acc_rows, name):
    rps = acc_rows // NSUB
    mesh = plsc.VectorSubcoreMesh(core_axis_name="c", subcore_axis_name="s")

    @functools.partial(
        pl.kernel,
        out_type=jax.ShapeDtypeStruct((2 * acc_rows, D), jnp.float32),
        mesh=mesh,
        scratch_types=[
            pltpu.VMEM((IB, CH), jnp.int32),
            pltpu.VMEM((IB, CH), jnp.int32),
            pltpu.VMEM((CH, D), jnp.float32),
            pltpu.VMEM((16, D), jnp.float32),
            pltpu.VMEM_SHARED((acc_rows, D), jnp.float32),
            pltpu.SemaphoreType.DMA,
        ],
        name=name,
        compiler_params=pltpu.CompilerParams(use_tc_tiling_on_sc=False),
    )
    def sc_stage(gidx_hbm, sidx_hbm, table_hbm, out_hbm,
                 gidx_v, sidx_v, rows_v, zb_v, acc_sh, sem):
        c = lax.axis_index("c")
        s = lax.axis_index("s")
        w = c * NSUB + s

        nz = D // 16
        def zrow(i, _):
            zb_v[i // nz, pl.ds((i % nz) * 16, 16)] = jnp.zeros((16,), jnp.float32)
            return 0
        lax.fori_loop(0, 16 * nz, zrow, 0)

        def zacc(t, _):
            pltpu.sync_copy(zb_v, acc_sh.at[pl.ds(s * rps + t * 16, 16)])
            return 0
        lax.fori_loop(0, rps // 16, zacc, 0)
        plsc.subcore_barrier()

        def block(b, _):
            pltpu.sync_copy(gidx_hbm.at[w * NB + b], gidx_v)
            pltpu.sync_copy(sidx_hbm.at[w * NB + b], sidx_v)

            def chunk(j, _):
                pltpu.async_copy(table_hbm.at[gidx_v.at[j]], rows_v, sem).wait()
                pltpu.sync_copy(rows_v, acc_sh.at[sidx_v.at[j]], add=True)
                return 0
            lax.fori_loop(0, IB, chunk, 0)
            return 0
        lax.fori_loop(0, NB, block, 0)
        plsc.subcore_barrier()

        def outcp(t, _):
            pltpu.sync_copy(acc_sh.at[pl.ds(s * rps + t * CH, CH)],
                            out_hbm.at[pl.ds(c * acc_rows + s * rps + t * CH, CH)])
            return 0
        lax.fori_loop(0, rps // CH, outcp, 0)

    return sc_stage


_sc_a1 = _make_sc_stage(D1, SPE, "sc_edge_agg1")
_sc_c1 = _make_sc_stage(D1, NP, "sc_vert_agg1")
_sc_a2 = _make_sc_stage(D2, SPE, "sc_edge_agg2")
_sc_c2 = _make_sc_stage(D2, NP, "sc_vert_agg2")

BLK = 1000
EBLK = 512


def _mm1_kernel(x_ref, w_ref, o_ref):
    y = jnp.dot(x_ref[...], w_ref[...], preferred_element_type=jnp.float32)
    col = lax.broadcasted_iota(jnp.int32, (BLK, 16), 1)
    aug = jnp.where(col == 0, 1.0, 0.0).astype(jnp.float32)
    o_ref[...] = jnp.concatenate([y, aug], axis=1)


def _scale1_kernel(se_ref, homo_ref, o_ref):
    se = se_ref[0]
    h = homo_ref[...]
    cnt = se[:, NHID:NHID + 1]
    xe = se[:, :NHID] * (h / jnp.clip(cnt, 1.0, None))
    col = lax.broadcasted_iota(jnp.int32, (EBLK, 16), 1)
    aug = jnp.where(col == 0, h, jnp.where(col == 1, 1.0, 0.0))
    o_ref[0] = jnp.concatenate([xe, aug], axis=1)


def _d1mm2_kernel(y1a_ref, svp_ref, wout_ref, z_ref, y2_ref):
    y1 = y1a_ref[:, :NHID]
    sv = svp_ref[0]
    att = sv[:, NHID:NHID + 1]
    cnt = sv[:, NHID + 1:NHID + 2]
    hv = 1.0 / (jnp.maximum(att, 1e-30) * jnp.clip(cnt, 1.0, None))
    x1 = jnp.maximum(y1 + sv[:, :NHID] * hv, 0.0)
    z_ref[0] = x1
    y2_ref[0] = jnp.dot(x1, wout_ref[...], preferred_element_type=jnp.float32)


def _scale2_kernel(se2_ref, se1_ref, homo_ref, o_ref):
    h = homo_ref[...]
    cnt = se1_ref[0][:, NHID:NHID + 1]
    o_ref[0] = se2_ref[0] * (h / jnp.clip(cnt, 1.0, None))


def _d2_kernel(y2_ref, sv2_ref, sv1_ref, o_ref):
    sv1 = sv1_ref[0]
    att = sv1[:, NHID:NHID + 1]
    cnt = sv1[:, NHID + 1:NHID + 2]
    hv = 1.0 / (jnp.maximum(att, 1e-30) * jnp.clip(cnt, 1.0, None))
    o_ref[0] = y2_ref[0] + sv2_ref[0] * hv


def kernel(X, W1, Wout, homo, H1_V, H1_E, H2_V, H2_E):
    f32 = jnp.float32

    Y1a = pl.pallas_call(
        _mm1_kernel, grid=(N // BLK,),
        in_specs=[pl.BlockSpec((BLK, NFEAT), lambda i: (i, 0)),
                  pl.BlockSpec((NFEAT, NHID), lambda i: (0, 0))],
        out_specs=pl.BlockSpec((BLK, D1), lambda i: (i, 0)),
        out_shape=jax.ShapeDtypeStruct((N, D1), f32),
    )(X, W1)

    def rs2(a, b):
        return jnp.stack([a.reshape(NSUB, NB, IB, CH),
                          b.reshape(NSUB, NB, IB, CH)]).reshape(2 * NSUB * NB, IB, CH)

    V_plain = rs2(H1_V, H2_V)
    E_plain = rs2(H1_E, H2_E)
    E_off = rs2(H1_E, H2_E + SPE)
    V_off = rs2(H1_V, H2_V + N)

    homo_p = jnp.pad(homo, (0, SPE - NE)).reshape(SPE, 1)

    Se1 = _sc_a1(V_plain, E_plain, Y1a)
    Se1v = Se1.reshape(2, SPE, D1)

    Xe1a = pl.pallas_call(
        _scale1_kernel, grid=(2, SPE // EBLK),
        in_specs=[pl.BlockSpec((1, EBLK, D1), lambda v, i: (v, i, 0)),
                  pl.BlockSpec((EBLK, 1), lambda v, i: (i, 0))],
        out_specs=pl.BlockSpec((1, EBLK, D1), lambda v, i: (v, i, 0)),
        out_shape=jax.ShapeDtypeStruct((2, SPE, D1), f32),
    )(Se1v, homo_p)

    Sv1 = _sc_c1(E_off, V_plain, Xe1a.reshape(2 * SPE, D1))
    Sv1v = Sv1.reshape(2, NP, D1)

    Z, Y2 = pl.pallas_call(
        _d1mm2_kernel, grid=(2, N // BLK),
        in_specs=[pl.BlockSpec((BLK, D1), lambda v, i: (i, 0)),
                  pl.BlockSpec((1, BLK, D1), lambda v, i: (v, i, 0)),
                  pl.BlockSpec((NHID, NCLASS), lambda v, i: (0, 0))],
        out_specs=[pl.BlockSpec((1, BLK, NHID), lambda v, i: (v, i, 0)),
                   pl.BlockSpec((1, BLK, NCLASS), lambda v, i: (v, i, 0))],
        out_shape=[jax.ShapeDtypeStruct((2, N, NHID), f32),
                   jax.ShapeDtypeStruct((2, N, NCLASS), f32)],
    )(Y1a, Sv1v, Wout)

    Se2 = _sc_a2(V_off, E_plain, Y2.reshape(2 * N, D2))

    Xe2 = pl.pallas_call(
        _scale2_kernel, grid=(2, SPE // EBLK),
        in_specs=[pl.BlockSpec((1, EBLK, D2), lambda v, i: (v, i, 0)),
                  pl.BlockSpec((1, EBLK, D1), lambda v, i: (v, i, 0)),
                  pl.BlockSpec((EBLK, 1), lambda v, i: (i, 0))],
        out_specs=pl.BlockSpec((1, EBLK, D2), lambda v, i: (v, i, 0)),
        out_shape=jax.ShapeDtypeStruct((2, SPE, D2), f32),
    )(Se2.reshape(2, SPE, D2), Se1v, homo_p)

    Sv2 = _sc_c2(E_off, V_plain, Xe2.reshape(2 * SPE, D2))

    Xo = pl.pallas_call(
        _d2_kernel, grid=(2, N // BLK),
        in_specs=[pl.BlockSpec((1, BLK, D2), lambda v, i: (v, i, 0)),
                  pl.BlockSpec((1, BLK, D2), lambda v, i: (v, i, 0)),
                  pl.BlockSpec((1, BLK, D1), lambda v, i: (v, i, 0))],
        out_specs=pl.BlockSpec((1, BLK, D2), lambda v, i: (v, i, 0)),
        out_shape=jax.ShapeDtypeStruct((2, N, D2), f32),
    )(Y2, Sv2.reshape(2, NP, D2), Sv1v)

    return (Z[0], Z[1], Xo[0], Xo[1])


# speedup vs baseline: 18.0860x; 1.6737x over previous
import functools

import jax
import jax.numpy as jnp
from jax import lax
from jax.experimental import pallas as pl
from jax.experimental.pallas import tpu as pltpu
from jax.experimental.pallas import tpu_sc as plsc

N = 10000
NE = 5000
INC = 320000
NFEAT = 128
NHID = 128
NCLASS = 64

D1 = NHID + 16
D2 = NCLASS
SPE = 5120
NP = 10240
NSUB = 16
PER_SUB = INC // NSUB
CH = 80
NCHUNK = PER_SUB // CH
IB = 5
NB = NCHUNK // IB
NSB = NB // 2


def _make_sc_stage(D, acc_rows, name):
    rps = acc_rows // NSUB
    mesh = plsc.VectorSubcoreMesh(core_axis_name="c", subcore_axis_name="s")

    @functools.partial(
        pl.kernel,
        out_type=jax.ShapeDtypeStruct((2 * acc_rows, D), jnp.float32),
        mesh=mesh,
        scratch_types=[
            pltpu.VMEM((IB, CH), jnp.int32),
            pltpu.VMEM((IB, CH), jnp.int32),
            pltpu.VMEM((IB, CH), jnp.int32),
            pltpu.VMEM((IB, CH), jnp.int32),
            pltpu.VMEM((CH, D), jnp.float32),
            pltpu.VMEM((CH, D), jnp.float32),
            pltpu.VMEM_SHARED((acc_rows, D), jnp.float32),
            pltpu.SemaphoreType.DMA,
            pltpu.SemaphoreType.DMA,
            pltpu.SemaphoreType.DMA,
            pltpu.SemaphoreType.DMA,
        ],
        name=name,
        compiler_params=pltpu.CompilerParams(use_tc_tiling_on_sc=False),
    )
    def sc_stage(gidx_hbm, sidx_hbm, table_hbm, out_hbm,
                 gidx0, gidx1, sidx0, sidx1, rows0, rows1, acc_sh,
                 sem_g, sem_s, sem_i0, sem_i1):
        c = lax.axis_index("c")
        s = lax.axis_index("s")
        w = c * NSUB + s
        wb = w * NB
        gidx = (gidx0, gidx1)
        sidx = (sidx0, sidx1)
        rows = (rows0, rows1)
        sem_i = (sem_i0, sem_i1)

        nz = D // 16
        def zrow(i, _):
            rows0[i // nz, pl.ds((i % nz) * 16, 16)] = jnp.zeros((16,), jnp.float32)
            return 0
        lax.fori_loop(0, CH * nz, zrow, 0)

        def zacc(t, _):
            pltpu.sync_copy(rows0, acc_sh.at[pl.ds(s * rps + t * CH, CH)])
            return 0
        lax.fori_loop(0, rps // CH, zacc, 0)
        plsc.subcore_barrier()

        def start_idx(block_id, p):
            pltpu.async_copy(gidx_hbm.at[block_id], gidx[p], sem_i[p])
            pltpu.async_copy(sidx_hbm.at[block_id], sidx[p], sem_i[p])

        def wait_idx(p):
            pltpu.make_async_copy(gidx_hbm.at[0], gidx[p], sem_i[p]).wait()
            pltpu.make_async_copy(sidx_hbm.at[0], sidx[p], sem_i[p]).wait()

        def start_gather(idx_ref, buf):
            pltpu.async_copy(table_hbm.at[idx_ref], buf, sem_g)

        def wait_gather(p):
            pltpu.make_async_copy(table_hbm.at[gidx0.at[0]], rows[p], sem_g).wait()

        def start_scatter(buf, idx_ref):
            pltpu.async_copy(buf, acc_sh.at[idx_ref], sem_s, add=True)

        def wait_scatter(p):
            pltpu.make_async_copy(rows[p], acc_sh.at[sidx0.at[0]], sem_s).wait()

        start_idx(wb, 0)
        wait_idx(0)
        start_gather(gidx0.at[0], rows0)

        def sb(g, _):
            not_last = g < NSB - 1
            for k in range(2 * IB):
                p = k % 2
                cur, nxt = rows[p], rows[1 - p]
                if k == 0:
                    @pl.when(g > 0)
                    def _():
                        wait_scatter(1 - p)
                else:
                    wait_scatter(1 - p)
                if k == 1:
                    start_idx(wb + 2 * g + 1, 1)
                if k == IB:
                    @pl.when(not_last)
                    def _():
                        start_idx(wb + 2 * (g + 1), 0)
                if k == IB - 1:
                    wait_idx(1)
                if k < IB - 1:
                    start_gather(gidx0.at[k + 1], nxt)
                elif k < 2 * IB - 1:
                    start_gather(gidx1.at[k + 1 - IB], nxt)
                else:
                    @pl.when(not_last)
                    def _():
                        wait_idx(0)
                        start_gather(gidx0.at[0], nxt)
                wait_gather(p)
                if k < IB:
                    start_scatter(cur, sidx0.at[k])
                else:
                    start_scatter(cur, sidx1.at[k - IB])
            return 0
        lax.fori_loop(0, NSB, sb, 0)
        wait_scatter(1)
        plsc.subcore_barrier()

        def outcp(t, _):
            pltpu.sync_copy(acc_sh.at[pl.ds(s * rps + t * CH, CH)],
                            out_hbm.at[pl.ds(c * acc_rows + s * rps + t * CH, CH)])
            return 0
        lax.fori_loop(0, rps // CH, outcp, 0)

    return sc_stage


_sc_a1 = _make_sc_stage(D1, SPE, "sc_edge_agg1")
_sc_c1 = _make_sc_stage(D1, NP, "sc_vert_agg1")
_sc_a2 = _make_sc_stage(D2, SPE, "sc_edge_agg2")
_sc_c2 = _make_sc_stage(D2, NP, "sc_vert_agg2")

BLK = 1000
EBLK = 512


def _mm1_kernel(x_ref, w_ref, o_ref):
    y = jnp.dot(x_ref[...], w_ref[...], preferred_element_type=jnp.float32)
    col = lax.broadcasted_iota(jnp.int32, (BLK, 16), 1)
    aug = jnp.where(col == 0, 1.0, 0.0).astype(jnp.float32)
    o_ref[...] = jnp.concatenate([y, aug], axis=1)


def _scale1_kernel(se_ref, homo_ref, o_ref):
    se = se_ref[0]
    h = homo_ref[...]
    cnt = se[:, NHID:NHID + 1]
    xe = se[:, :NHID] * (h / jnp.clip(cnt, 1.0, None))
    col = lax.broadcasted_iota(jnp.int32, (EBLK, 16), 1)
    aug = jnp.where(col == 0, h, jnp.where(col == 1, 1.0, 0.0))
    o_ref[0] = jnp.concatenate([xe, aug], axis=1)


def _d1mm2_kernel(y1a_ref, svp_ref, wout_ref, z_ref, y2_ref):
    y1 = y1a_ref[:, :NHID]
    sv = svp_ref[0]
    att = sv[:, NHID:NHID + 1]
    cnt = sv[:, NHID + 1:NHID + 2]
    hv = 1.0 / (jnp.maximum(att, 1e-30) * jnp.clip(cnt, 1.0, None))
    x1 = jnp.maximum(y1 + sv[:, :NHID] * hv, 0.0)
    z_ref[0] = x1
    y2_ref[0] = jnp.dot(x1, wout_ref[...], preferred_element_type=jnp.float32)


def _scale2_kernel(se2_ref, se1_ref, homo_ref, o_ref):
    h = homo_ref[...]
    cnt = se1_ref[0][:, NHID:NHID + 1]
    o_ref[0] = se2_ref[0] * (h / jnp.clip(cnt, 1.0, None))


def _d2_kernel(y2_ref, sv2_ref, sv1_ref, o_ref):
    sv1 = sv1_ref[0]
    att = sv1[:, NHID:NHID + 1]
    cnt = sv1[:, NHID + 1:NHID + 2]
    hv = 1.0 / (jnp.maximum(att, 1e-30) * jnp.clip(cnt, 1.0, None))
    o_ref[0] = y2_ref[0] + sv2_ref[0] * hv


def kernel(X, W1, Wout, homo, H1_V, H1_E, H2_V, H2_E):
    f32 = jnp.float32

    Y1a = pl.pallas_call(
        _mm1_kernel, grid=(N // BLK,),
        in_specs=[pl.BlockSpec((BLK, NFEAT), lambda i: (i, 0)),
                  pl.BlockSpec((NFEAT, NHID), lambda i: (0, 0))],
        out_specs=pl.BlockSpec((BLK, D1), lambda i: (i, 0)),
        out_shape=jax.ShapeDtypeStruct((N, D1), f32),
    )(X, W1)

    def rs2(a, b):
        return jnp.stack([a.reshape(NSUB, NB, IB, CH),
                          b.reshape(NSUB, NB, IB, CH)]).reshape(2 * NSUB * NB, IB, CH)

    V_plain = rs2(H1_V, H2_V)
    E_plain = rs2(H1_E, H2_E)
    E_off = rs2(H1_E, H2_E + SPE)
    V_off = rs2(H1_V, H2_V + N)

    homo_p = jnp.pad(homo, (0, SPE - NE)).reshape(SPE, 1)

    Se1 = _sc_a1(V_plain, E_plain, Y1a)
    Se1v = Se1.reshape(2, SPE, D1)

    Xe1a = pl.pallas_call(
        _scale1_kernel, grid=(2, SPE // EBLK),
        in_specs=[pl.BlockSpec((1, EBLK, D1), lambda v, i: (v, i, 0)),
                  pl.BlockSpec((EBLK, 1), lambda v, i: (i, 0))],
        out_specs=pl.BlockSpec((1, EBLK, D1), lambda v, i: (v, i, 0)),
        out_shape=jax.ShapeDtypeStruct((2, SPE, D1), f32),
    )(Se1v, homo_p)

    Sv1 = _sc_c1(E_off, V_plain, Xe1a.reshape(2 * SPE, D1))
    Sv1v = Sv1.reshape(2, NP, D1)

    Z, Y2 = pl.pallas_call(
        _d1mm2_kernel, grid=(2, N // BLK),
        in_specs=[pl.BlockSpec((BLK, D1), lambda v, i: (i, 0)),
                  pl.BlockSpec((1, BLK, D1), lambda v, i: (v, i, 0)),
                  pl.BlockSpec((NHID, NCLASS), lambda v, i: (0, 0))],
        out_specs=[pl.BlockSpec((1, BLK, NHID), lambda v, i: (v, i, 0)),
                   pl.BlockSpec((1, BLK, NCLASS), lambda v, i: (v, i, 0))],
        out_shape=[jax.ShapeDtypeStruct((2, N, NHID), f32),
                   jax.ShapeDtypeStruct((2, N, NCLASS), f32)],
    )(Y1a, Sv1v, Wout)

    Se2 = _sc_a2(V_off, E_plain, Y2.reshape(2 * N, D2))

    Xe2 = pl.pallas_call(
        _scale2_kernel, grid=(2, SPE // EBLK),
        in_specs=[pl.BlockSpec((1, EBLK, D2), lambda v, i: (v, i, 0)),
                  pl.BlockSpec((1, EBLK, D1), lambda v, i: (v, i, 0)),
                  pl.BlockSpec((EBLK, 1), lambda v, i: (i, 0))],
        out_specs=pl.BlockSpec((1, EBLK, D2), lambda v, i: (v, i, 0)),
        out_shape=jax.ShapeDtypeStruct((2, SPE, D2), f32),
    )(Se2.reshape(2, SPE, D2), Se1v, homo_p)

    Sv2 = _sc_c2(E_off, V_plain, Xe2.reshape(2 * SPE, D2))

    Xo = pl.pallas_call(
        _d2_kernel, grid=(2, N // BLK),
        in_specs=[pl.BlockSpec((1, BLK, D2), lambda v, i: (v, i, 0)),
                  pl.BlockSpec((1, BLK, D2), lambda v, i: (v, i, 0)),
                  pl.BlockSpec((1, BLK, D1), lambda v, i: (v, i, 0))],
        out_specs=pl.BlockSpec((1, BLK, D2), lambda v, i: (v, i, 0)),
        out_shape=jax.ShapeDtypeStruct((2, N, D2), f32),
    )(Y2, Sv2.reshape(2, NP, D2), Sv1v)

    return (Z[0], Z[1], Xo[0], Xo[1])


# speedup vs baseline: 20.2698x; 1.1207x over previous
import functools

import jax
import jax.numpy as jnp
from jax import lax
from jax.experimental import pallas as pl
from jax.experimental.pallas import tpu as pltpu
from jax.experimental.pallas import tpu_sc as plsc

N = 10000
NE = 5000
INC = 320000
NFEAT = 128
NHID = 128
NCLASS = 64

D1 = NHID + 16
D2 = NCLASS
SPE = 5120
NP = 10240
NSUB = 16
PER_SUB = INC // NSUB
CH = 125
NCHUNK = PER_SUB // CH
IB = 5
NB = NCHUNK // IB
NSB = NB // 2
OC = 80


def _make_sc_stage(D, acc_rows, name):
    rps = acc_rows // NSUB
    mesh = plsc.VectorSubcoreMesh(core_axis_name="c", subcore_axis_name="s")

    @functools.partial(
        pl.kernel,
        out_type=jax.ShapeDtypeStruct((2 * acc_rows, D), jnp.float32),
        mesh=mesh,
        scratch_types=[
            pltpu.VMEM((IB, CH), jnp.int32),
            pltpu.VMEM((IB, CH), jnp.int32),
            pltpu.VMEM((IB, CH), jnp.int32),
            pltpu.VMEM((IB, CH), jnp.int32),
            pltpu.VMEM((CH, D), jnp.float32),
            pltpu.VMEM((CH, D), jnp.float32),
            pltpu.VMEM_SHARED((acc_rows, D), jnp.float32),
            pltpu.SemaphoreType.DMA,
            pltpu.SemaphoreType.DMA,
            pltpu.SemaphoreType.DMA,
            pltpu.SemaphoreType.DMA,
        ],
        name=name,
        compiler_params=pltpu.CompilerParams(use_tc_tiling_on_sc=False),
    )
    def sc_stage(gidx_hbm, sidx_hbm, table_hbm, out_hbm,
                 gidx0, gidx1, sidx0, sidx1, rows0, rows1, acc_sh,
                 sem_g, sem_s, sem_i0, sem_i1):
        c = lax.axis_index("c")
        s = lax.axis_index("s")
        w = c * NSUB + s
        wb = w * NB
        gidx = (gidx0, gidx1)
        sidx = (sidx0, sidx1)
        rows = (rows0, rows1)
        sem_i = (sem_i0, sem_i1)

        nz = D // 16
        def zrow(i, _):
            rows0[i // nz, pl.ds((i % nz) * 16, 16)] = jnp.zeros((16,), jnp.float32)
            return 0
        lax.fori_loop(0, OC * nz, zrow, 0)

        def zacc(t, _):
            pltpu.sync_copy(rows0.at[pl.ds(0, OC)],
                            acc_sh.at[pl.ds(s * rps + t * OC, OC)])
            return 0
        lax.fori_loop(0, rps // OC, zacc, 0)
        plsc.subcore_barrier()

        def start_idx(block_id, p):
            pltpu.async_copy(gidx_hbm.at[block_id], gidx[p], sem_i[p])
            pltpu.async_copy(sidx_hbm.at[block_id], sidx[p], sem_i[p])

        def wait_idx(p):
            pltpu.make_async_copy(gidx_hbm.at[0], gidx[p], sem_i[p]).wait()
            pltpu.make_async_copy(sidx_hbm.at[0], sidx[p], sem_i[p]).wait()

        def start_gather(idx_ref, buf):
            pltpu.async_copy(table_hbm.at[idx_ref], buf, sem_g)

        def wait_gather(p):
            pltpu.make_async_copy(table_hbm.at[gidx0.at[0]], rows[p], sem_g).wait()

        def start_scatter(buf, idx_ref):
            pltpu.async_copy(buf, acc_sh.at[idx_ref], sem_s, add=True)

        def wait_scatter(p):
            pltpu.make_async_copy(rows[p], acc_sh.at[sidx0.at[0]], sem_s).wait()

        start_idx(wb, 0)
        wait_idx(0)
        start_gather(gidx0.at[0], rows0)

        def sb(g, _):
            not_last = g < NSB - 1
            for k in range(2 * IB):
                p = k % 2
                cur, nxt = rows[p], rows[1 - p]
                if k == 0:
                    @pl.when(g > 0)
                    def _():
                        wait_scatter(1 - p)
                else:
                    wait_scatter(1 - p)
                if k == 1:
                    start_idx(wb + 2 * g + 1, 1)
                if k == IB:
                    @pl.when(not_last)
                    def _():
                        start_idx(wb + 2 * (g + 1), 0)
                if k == IB - 1:
                    wait_idx(1)
                if k < IB - 1:
                    start_gather(gidx0.at[k + 1], nxt)
                elif k < 2 * IB - 1:
                    start_gather(gidx1.at[k + 1 - IB], nxt)
                else:
                    @pl.when(not_last)
                    def _():
                        wait_idx(0)
                        start_gather(gidx0.at[0], nxt)
                wait_gather(p)
                if k < IB:
                    start_scatter(cur, sidx0.at[k])
                else:
                    start_scatter(cur, sidx1.at[k - IB])
            return 0
        lax.fori_loop(0, NSB, sb, 0)
        wait_scatter(1)
        plsc.subcore_barrier()

        def outcp(t, _):
            pltpu.sync_copy(acc_sh.at[pl.ds(s * rps + t * OC, OC)],
                            out_hbm.at[pl.ds(c * acc_rows + s * rps + t * OC, OC)])
            return 0
        lax.fori_loop(0, rps // OC, outcp, 0)

    return sc_stage


_sc_a1 = _make_sc_stage(D1, SPE, "sc_edge_agg1")
_sc_c1 = _make_sc_stage(D1, NP, "sc_vert_agg1")
_sc_a2 = _make_sc_stage(D2, SPE, "sc_edge_agg2")
_sc_c2 = _make_sc_stage(D2, NP, "sc_vert_agg2")

BLK = 1000
EBLK = 512


def _mm1_kernel(x_ref, w_ref, o_ref):
    y = jnp.dot(x_ref[...], w_ref[...], preferred_element_type=jnp.float32)
    col = lax.broadcasted_iota(jnp.int32, (BLK, 16), 1)
    aug = jnp.where(col == 0, 1.0, 0.0).astype(jnp.float32)
    o_ref[...] = jnp.concatenate([y, aug], axis=1)


def _scale1_kernel(se_ref, homo_ref, o_ref):
    se = se_ref[0]
    h = homo_ref[...]
    cnt = se[:, NHID:NHID + 1]
    xe = se[:, :NHID] * (h / jnp.clip(cnt, 1.0, None))
    col = lax.broadcasted_iota(jnp.int32, (EBLK, 16), 1)
    aug = jnp.where(col == 0, h, jnp.where(col == 1, 1.0, 0.0))
    o_ref[0] = jnp.concatenate([xe, aug], axis=1)


def _d1mm2_kernel(y1a_ref, svp_ref, wout_ref, z_ref, y2_ref):
    y1 = y1a_ref[:, :NHID]
    sv = svp_ref[0]
    att = sv[:, NHID:NHID + 1]
    cnt = sv[:, NHID + 1:NHID + 2]
    hv = 1.0 / (jnp.maximum(att, 1e-30) * jnp.clip(cnt, 1.0, None))
    x1 = jnp.maximum(y1 + sv[:, :NHID] * hv, 0.0)
    z_ref[0] = x1
    y2_ref[0] = jnp.dot(x1, wout_ref[...], preferred_element_type=jnp.float32)


def _scale2_kernel(se2_ref, se1_ref, homo_ref, o_ref):
    h = homo_ref[...]
    cnt = se1_ref[0][:, NHID:NHID + 1]
    o_ref[0] = se2_ref[0] * (h / jnp.clip(cnt, 1.0, None))


def _d2_kernel(y2_ref, sv2_ref, sv1_ref, o_ref):
    sv1 = sv1_ref[0]
    att = sv1[:, NHID:NHID + 1]
    cnt = sv1[:, NHID + 1:NHID + 2]
    hv = 1.0 / (jnp.maximum(att, 1e-30) * jnp.clip(cnt, 1.0, None))
    o_ref[0] = y2_ref[0] + sv2_ref[0] * hv


def kernel(X, W1, Wout, homo, H1_V, H1_E, H2_V, H2_E):
    f32 = jnp.float32

    Y1a = pl.pallas_call(
        _mm1_kernel, grid=(N // BLK,),
        in_specs=[pl.BlockSpec((BLK, NFEAT), lambda i: (i, 0)),
                  pl.BlockSpec((NFEAT, NHID), lambda i: (0, 0))],
        out_specs=pl.BlockSpec((BLK, D1), lambda i: (i, 0)),
        out_shape=jax.ShapeDtypeStruct((N, D1), f32),
    )(X, W1)

    def rs2(a, b):
        return jnp.stack([a.reshape(NSUB, NB, IB, CH),
                          b.reshape(NSUB, NB, IB, CH)]).reshape(2 * NSUB * NB, IB, CH)

    V_plain = rs2(H1_V, H2_V)
    E_plain = rs2(H1_E, H2_E)
    E_off = rs2(H1_E, H2_E + SPE)
    V_off = rs2(H1_V, H2_V + N)

    homo_p = jnp.pad(homo, (0, SPE - NE)).reshape(SPE, 1)

    Se1 = _sc_a1(V_plain, E_plain, Y1a)
    Se1v = Se1.reshape(2, SPE, D1)

    Xe1a = pl.pallas_call(
        _scale1_kernel, grid=(2, SPE // EBLK),
        in_specs=[pl.BlockSpec((1, EBLK, D1), lambda v, i: (v, i, 0)),
                  pl.BlockSpec((EBLK, 1), lambda v, i: (i, 0))],
        out_specs=pl.BlockSpec((1, EBLK, D1), lambda v, i: (v, i, 0)),
        out_shape=jax.ShapeDtypeStruct((2, SPE, D1), f32),
    )(Se1v, homo_p)

    Sv1 = _sc_c1(E_off, V_plain, Xe1a.reshape(2 * SPE, D1))
    Sv1v = Sv1.reshape(2, NP, D1)

    Z, Y2 = pl.pallas_call(
        _d1mm2_kernel, grid=(2, N // BLK),
        in_specs=[pl.BlockSpec((BLK, D1), lambda v, i: (i, 0)),
                  pl.BlockSpec((1, BLK, D1), lambda v, i: (v, i, 0)),
                  pl.BlockSpec((NHID, NCLASS), lambda v, i: (0, 0))],
        out_specs=[pl.BlockSpec((1, BLK, NHID), lambda v, i: (v, i, 0)),
                   pl.BlockSpec((1, BLK, NCLASS), lambda v, i: (v, i, 0))],
        out_shape=[jax.ShapeDtypeStruct((2, N, NHID), f32),
                   jax.ShapeDtypeStruct((2, N, NCLASS), f32)],
    )(Y1a, Sv1v, Wout)

    Se2 = _sc_a2(V_off, E_plain, Y2.reshape(2 * N, D2))

    Xe2 = pl.pallas_call(
        _scale2_kernel, grid=(2, SPE // EBLK),
        in_specs=[pl.BlockSpec((1, EBLK, D2), lambda v, i: (v, i, 0)),
                  pl.BlockSpec((1, EBLK, D1), lambda v, i: (v, i, 0)),
                  pl.BlockSpec((EBLK, 1), lambda v, i: (i, 0))],
        out_specs=pl.BlockSpec((1, EBLK, D2), lambda v, i: (v, i, 0)),
        out_shape=jax.ShapeDtypeStruct((2, SPE, D2), f32),
    )(Se2.reshape(2, SPE, D2), Se1v, homo_p)

    Sv2 = _sc_c2(E_off, V_plain, Xe2.reshape(2 * SPE, D2))

    Xo = pl.pallas_call(
        _d2_kernel, grid=(2, N // BLK),
        in_specs=[pl.BlockSpec((1, BLK, D2), lambda v, i: (v, i, 0)),
                  pl.BlockSpec((1, BLK, D2), lambda v, i: (v, i, 0)),
                  pl.BlockSpec((1, BLK, D1), lambda v, i: (v, i, 0))],
        out_specs=pl.BlockSpec((1, BLK, D2), lambda v, i: (v, i, 0)),
        out_shape=jax.ShapeDtypeStruct((2, N, D2), f32),
    )(Y2, Sv2.reshape(2, NP, D2), Sv1v)

    return (Z[0], Z[1], Xo[0], Xo[1])


# speedup vs baseline: 21.5909x; 1.0652x over previous
import functools

import jax
import jax.numpy as jnp
from jax import lax
from jax.experimental import pallas as pl
from jax.experimental.pallas import tpu as pltpu
from jax.experimental.pallas import tpu_sc as plsc

N = 10000
NE = 5000
INC = 320000
NFEAT = 128
NHID = 128
NCLASS = 64

D1 = NHID + 16
D2 = NCLASS
SPE = 5120
NP = 10240
NSUB = 16
PER_SUB = INC // NSUB
CH = 125
NCHUNK = PER_SUB // CH
IB = 5
NB = NCHUNK // IB
NSB = NB // 2
OC = 80


def _bc(v, r):
    return v.at[jnp.full((16,), r, jnp.int32)].get(mode="promise_in_bounds")


def _make_sc_stage(D, acc_rows, name, mode):
    rps = acc_rows // NSUB
    mesh = plsc.VectorSubcoreMesh(core_axis_name="c", subcore_axis_name="s")

    out_type = jax.ShapeDtypeStruct((2 * acc_rows, D), jnp.float32)
    if mode == "a1":
        out_type = [out_type, jax.ShapeDtypeStruct((2 * acc_rows,), jnp.float32)]
    scratch = [
        pltpu.VMEM((IB, CH), jnp.int32),
        pltpu.VMEM((IB, CH), jnp.int32),
        pltpu.VMEM((IB, CH), jnp.int32),
        pltpu.VMEM((IB, CH), jnp.int32),
        pltpu.VMEM((CH, D), jnp.float32),
        pltpu.VMEM((CH, D), jnp.float32),
        pltpu.VMEM((OC,), jnp.float32),
        pltpu.VMEM((OC,), jnp.float32),
        pltpu.VMEM_SHARED((acc_rows, D), jnp.float32),
        pltpu.SemaphoreType.DMA,
        pltpu.SemaphoreType.DMA,
        pltpu.SemaphoreType.DMA,
        pltpu.SemaphoreType.DMA,
    ]

    def sc_body(refs):
        (gidx_hbm, sidx_hbm, table_hbm, sval_hbm, out_hbm, gout_hbm,
         gidx0, gidx1, sidx0, sidx1, rows0, rows1, sval_v, g_v, acc_sh,
         sem_g, sem_s, sem_i0, sem_i1) = refs
        c = lax.axis_index("c")
        s = lax.axis_index("s")
        gidx = (gidx0, gidx1)
        sidx = (sidx0, sidx1)
        rows = (rows0, rows1)
        sem_i = (sem_i0, sem_i1)

        nz = D // 16
        def zrow(i, _):
            rows0[i // nz, pl.ds((i % nz) * 16, 16)] = jnp.zeros((16,), jnp.float32)
            return 0
        lax.fori_loop(0, OC * nz, zrow, 0)

        def zacc(t, _):
            pltpu.sync_copy(rows0.at[pl.ds(0, OC)],
                            acc_sh.at[pl.ds(s * rps + t * OC, OC)])
            return 0
        lax.fori_loop(0, rps // OC, zacc, 0)
        plsc.subcore_barrier()

        def start_idx(b, p):
            blk = (c * NSUB + s) * NB + b
            pltpu.async_copy(gidx_hbm.at[blk], gidx[p], sem_i[p])
            pltpu.async_copy(sidx_hbm.at[blk], sidx[p], sem_i[p])

        def wait_idx(p):
            pltpu.make_async_copy(gidx_hbm.at[0], gidx[p], sem_i[p]).wait()
            pltpu.make_async_copy(sidx_hbm.at[0], sidx[p], sem_i[p]).wait()

        def start_gather(idx_ref, buf):
            pltpu.async_copy(table_hbm.at[idx_ref], buf, sem_g)

        def wait_gather(p):
            pltpu.make_async_copy(table_hbm.at[gidx0.at[0]], rows[p], sem_g).wait()

        def start_scatter(buf, idx_ref):
            pltpu.async_copy(buf, acc_sh.at[idx_ref], sem_s, add=True)

        def wait_scatter(p):
            pltpu.make_async_copy(rows[p], acc_sh.at[sidx0.at[0]], sem_s).wait()

        start_idx(0, 0)
        wait_idx(0)
        start_gather(gidx0.at[0], rows0)

        def sb(g, _):
            not_last = g < NSB - 1
            for k in range(2 * IB):
                p = k % 2
                cur, nxt = rows[p], rows[1 - p]
                if k == 0:
                    @pl.when(g > 0)
                    def _():
                        wait_scatter(1 - p)
                else:
                    wait_scatter(1 - p)
                if k == 1:
                    start_idx(2 * g + 1, 1)
                if k == IB:
                    @pl.when(not_last)
                    def _():
                        start_idx(2 * (g + 1), 0)
                if k == IB - 1:
                    wait_idx(1)
                if k < IB - 1:
                    start_gather(gidx0.at[k + 1], nxt)
                elif k < 2 * IB - 1:
                    start_gather(gidx1.at[k + 1 - IB], nxt)
                else:
                    @pl.when(not_last)
                    def _():
                        wait_idx(0)
                        start_gather(gidx0.at[0], nxt)
                wait_gather(p)
                if k < IB:
                    start_scatter(cur, sidx0.at[k])
                else:
                    start_scatter(cur, sidx1.at[k - IB])
            return 0
        lax.fori_loop(0, NSB, sb, 0)
        wait_scatter(1)
        plsc.subcore_barrier()

        i16 = lax.broadcasted_iota(jnp.int32, (16,), 0)
        nd = D // 16

        def outcp(t, _):
            base = s * rps + t * OC
            obase = c * acc_rows + base
            if mode == "c":
                pltpu.sync_copy(acc_sh.at[pl.ds(base, OC)],
                                out_hbm.at[pl.ds(obase, OC)])
                return 0
            pltpu.sync_copy(acc_sh.at[pl.ds(base, OC)], rows0.at[pl.ds(0, OC)])
            if mode == "a1":
                pltpu.sync_copy(sval_hbm.at[pl.ds(base, OC)], sval_v)
            else:
                pltpu.sync_copy(sval_hbm.at[pl.ds(obase, OC)], sval_v)
            for q in range(OC // 16):
                rb = q * 16
                if mode == "a1":
                    homo16 = sval_v[pl.ds(rb, 16)]
                    cnt16 = plsc.load_gather(
                        rows0, [rb + i16, jnp.full((16,), NHID, jnp.int32)])
                    scale16 = homo16 / jnp.maximum(cnt16, 1.0)
                    g_v[pl.ds(rb, 16)] = scale16
                else:
                    scale16 = sval_v[pl.ds(rb, 16)]
                for r in range(16):
                    row = rb + r
                    sc16 = _bc(scale16, r)
                    nfeat_regs = (NHID // 16) if mode == "a1" else nd
                    for k2 in range(nfeat_regs):
                        rows0[row, pl.ds(k2 * 16, 16)] = (
                            rows0[row, pl.ds(k2 * 16, 16)] * sc16)
                    if mode == "a1":
                        h16 = _bc(homo16, r)
                        rows0[row, pl.ds(NHID, 16)] = jnp.where(
                            i16 == 0, h16,
                            jnp.where(i16 == 1, jnp.float32(1.0), jnp.float32(0.0)))
            pltpu.sync_copy(rows0.at[pl.ds(0, OC)], out_hbm.at[pl.ds(obase, OC)])
            if mode == "a1":
                pltpu.sync_copy(g_v, gout_hbm.at[pl.ds(obase, OC)])
            return 0
        lax.fori_loop(0, rps // OC, outcp, 0)

    kern_name = name

    if mode == "a1":
        @functools.partial(pl.kernel, out_type=out_type, mesh=mesh,
                           scratch_types=scratch, name=kern_name,
                           compiler_params=pltpu.CompilerParams(use_tc_tiling_on_sc=False, needs_layout_passes=False))
        def sc_stage(gI, sI, table, homo_hbm, out, gout, *scr):
            sc_body((gI, sI, table, homo_hbm, out, gout) + scr)
    elif mode == "a2":
        @functools.partial(pl.kernel, out_type=out_type, mesh=mesh,
                           scratch_types=scratch, name=kern_name,
                           compiler_params=pltpu.CompilerParams(use_tc_tiling_on_sc=False, needs_layout_passes=False))
        def sc_stage(gI, sI, table, g_hbm, out, *scr):
            sc_body((gI, sI, table, g_hbm, out, None) + scr)
    else:
        @functools.partial(pl.kernel, out_type=out_type, mesh=mesh,
                           scratch_types=scratch, name=kern_name,
                           compiler_params=pltpu.CompilerParams(use_tc_tiling_on_sc=False, needs_layout_passes=False))
        def sc_stage(gI, sI, table, out, *scr):
            sc_body((gI, sI, table, None, out, None) + scr)

    return sc_stage


_sc_a1 = _make_sc_stage(D1, SPE, "sc_edge_agg1", "a1")
_sc_c1 = _make_sc_stage(D1, NP, "sc_vert_agg1", "c")
_sc_a2 = _make_sc_stage(D2, SPE, "sc_edge_agg2", "a2")
_sc_c2 = _make_sc_stage(D2, NP, "sc_vert_agg2", "c")

BLK = 1000
EBLK = 512


def _mm1_kernel(x_ref, w_ref, o_ref):
    y = jnp.dot(x_ref[...], w_ref[...], preferred_element_type=jnp.float32)
    col = lax.broadcasted_iota(jnp.int32, (BLK, 16), 1)
    aug = jnp.where(col == 0, 1.0, 0.0).astype(jnp.float32)
    o_ref[...] = jnp.concatenate([y, aug], axis=1)


def _d1mm2_kernel(y1a_ref, svp_ref, wout_ref, z_ref, y2_ref):
    y1 = y1a_ref[:, :NHID]
    sv = svp_ref[0]
    att = sv[:, NHID:NHID + 1]
    cnt = sv[:, NHID + 1:NHID + 2]
    hv = 1.0 / (jnp.maximum(att, 1e-30) * jnp.clip(cnt, 1.0, None))
    x1 = jnp.maximum(y1 + sv[:, :NHID] * hv, 0.0)
    z_ref[0] = x1
    y2_ref[0] = jnp.dot(x1, wout_ref[...], preferred_element_type=jnp.float32)


def _d2_kernel(y2_ref, sv2_ref, sv1_ref, o_ref):
    sv1 = sv1_ref[0]
    att = sv1[:, NHID:NHID + 1]
    cnt = sv1[:, NHID + 1:NHID + 2]
    hv = 1.0 / (jnp.maximum(att, 1e-30) * jnp.clip(cnt, 1.0, None))
    o_ref[0] = y2_ref[0] + sv2_ref[0] * hv


def kernel(X, W1, Wout, homo, H1_V, H1_E, H2_V, H2_E):
    f32 = jnp.float32

    Y1a = pl.pallas_call(
        _mm1_kernel, grid=(N // BLK,),
        in_specs=[pl.BlockSpec((BLK, NFEAT), lambda i: (i, 0)),
                  pl.BlockSpec((NFEAT, NHID), lambda i: (0, 0))],
        out_specs=pl.BlockSpec((BLK, D1), lambda i: (i, 0)),
        out_shape=jax.ShapeDtypeStruct((N, D1), f32),
    )(X, W1)

    def rs2(a, b):
        return jnp.concatenate([a.reshape(NSUB * NB, IB, CH),
                                b.reshape(NSUB * NB, IB, CH)])

    V_plain = rs2(H1_V, H2_V)
    E_plain = rs2(H1_E, H2_E)
    E_off = rs2(H1_E, H2_E + SPE)
    V_off = rs2(H1_V, H2_V + N)

    homo_p = jnp.pad(homo, (0, SPE - NE))

    Xe1a, gvec = _sc_a1(V_plain, E_plain, Y1a, homo_p)

    Sv1 = _sc_c1(E_off, V_plain, Xe1a)
    Sv1v = Sv1.reshape(2, NP, D1)

    Z, Y2 = pl.pallas_call(
        _d1mm2_kernel, grid=(2, N // BLK),
        in_specs=[pl.BlockSpec((BLK, D1), lambda v, i: (i, 0)),
                  pl.BlockSpec((1, BLK, D1), lambda v, i: (v, i, 0)),
                  pl.BlockSpec((NHID, NCLASS), lambda v, i: (0, 0))],
        out_specs=[pl.BlockSpec((1, BLK, NHID), lambda v, i: (v, i, 0)),
                   pl.BlockSpec((1, BLK, NCLASS), lambda v, i: (v, i, 0))],
        out_shape=[jax.ShapeDtypeStruct((2, N, NHID), f32),
                   jax.ShapeDtypeStruct((2, N, NCLASS), f32)],
    )(Y1a, Sv1v, Wout)

    Xe2 = _sc_a2(V_off, E_plain, Y2.reshape(2 * N, D2), gvec)

    Sv2 = _sc_c2(E_off, V_plain, Xe2)

    Xo = pl.pallas_call(
        _d2_kernel, grid=(2, N // BLK),
        in_specs=[pl.BlockSpec((1, BLK, D2), lambda v, i: (v, i, 0)),
                  pl.BlockSpec((1, BLK, D2), lambda v, i: (v, i, 0)),
                  pl.BlockSpec((1, BLK, D1), lambda v, i: (v, i, 0))],
        out_specs=pl.BlockSpec((1, BLK, D2), lambda v, i: (v, i, 0)),
        out_shape=jax.ShapeDtypeStruct((2, N, D2), f32),
    )(Y2, Sv2.reshape(2, NP, D2), Sv1v)

    return (Z[0], Z[1], Xo[0], Xo[1])


# speedup vs baseline: 21.6902x; 1.0046x over previous
import functools

import jax
import jax.numpy as jnp
from jax import lax
from jax.experimental import pallas as pl
from jax.experimental.pallas import tpu as pltpu
from jax.experimental.pallas import tpu_sc as plsc

N = 10000
NE = 5000
INC = 320000
NFEAT = 128
NHID = 128
NCLASS = 64

D1 = NHID + 16
D2 = NCLASS
SPE = 5120
NP = 10240
NSUB = 16
PER_SUB = INC // NSUB
CH = 125
NCHUNK = PER_SUB // CH
IB = 5
NB = NCHUNK // IB
NSB = NB // 2
OC = 80


def _bc(v, r):
    return v.at[jnp.full((16,), r, jnp.int32)].get(mode="promise_in_bounds")


def _make_sc_stage(D, acc_rows, name, mode, split_idx=False):
    rps = acc_rows // NSUB
    mesh = plsc.VectorSubcoreMesh(core_axis_name="c", subcore_axis_name="s")
    IR = 2 * IB if split_idx else IB

    out_type = jax.ShapeDtypeStruct((2 * acc_rows, D), jnp.float32)
    if mode == "a1":
        out_type = [out_type, jax.ShapeDtypeStruct((2 * acc_rows,), jnp.float32)]
    scratch = [
        pltpu.VMEM((IR, CH), jnp.int32),
        pltpu.VMEM((IR, CH), jnp.int32),
        pltpu.VMEM((IR, CH), jnp.int32),
        pltpu.VMEM((IR, CH), jnp.int32),
        pltpu.VMEM((CH, D), jnp.float32),
        pltpu.VMEM((CH, D), jnp.float32),
        pltpu.VMEM((OC,), jnp.float32),
        pltpu.VMEM((OC,), jnp.float32),
        pltpu.VMEM_SHARED((acc_rows, D), jnp.float32),
        pltpu.SemaphoreType.DMA,
        pltpu.SemaphoreType.DMA,
        pltpu.SemaphoreType.DMA,
        pltpu.SemaphoreType.DMA,
    ]

    def sc_body(refs):
        (gidx_hbm, gidx_hbmB, sidx_hbm, sidx_hbmB, table_hbm, sval_hbm,
         sval2_hbm, out_hbm, gout_hbm,
         gidx0, gidx1, sidx0, sidx1, rows0, rows1, sval_v, g_v, acc_sh,
         sem_g, sem_s, sem_i0, sem_i1) = refs
        c = lax.axis_index("c")
        s = lax.axis_index("s")
        gidx = (gidx0, gidx1)
        sidx = (sidx0, sidx1)
        rows = (rows0, rows1)
        sem_i = (sem_i0, sem_i1)

        nz = D // 16
        def zrow(i, _):
            rows0[i // nz, pl.ds((i % nz) * 16, 16)] = jnp.zeros((16,), jnp.float32)
            return 0
        lax.fori_loop(0, OC * nz, zrow, 0)

        def zacc(t, _):
            pltpu.sync_copy(rows0.at[pl.ds(0, OC)],
                            acc_sh.at[pl.ds(s * rps + t * OC, OC)])
            return 0
        lax.fori_loop(0, rps // OC, zacc, 0)
        plsc.subcore_barrier()

        if split_idx:
            def start_idx(b, p):
                blk = s * NB + b
                pltpu.async_copy(gidx_hbm.at[blk], gidx[p].at[pl.ds(0, IB)], sem_i[p])
                pltpu.async_copy(gidx_hbmB.at[blk], gidx[p].at[pl.ds(IB, IB)], sem_i[p])
                pltpu.async_copy(sidx_hbm.at[blk], sidx[p].at[pl.ds(0, IB)], sem_i[p])
                pltpu.async_copy(sidx_hbmB.at[blk], sidx[p].at[pl.ds(IB, IB)], sem_i[p])

            def wait_idx(p):
                for _ in range(2):
                    pltpu.make_async_copy(gidx_hbm.at[0], gidx[p].at[pl.ds(0, IB)], sem_i[p]).wait()
                    pltpu.make_async_copy(sidx_hbm.at[0], sidx[p].at[pl.ds(0, IB)], sem_i[p]).wait()

            def gref(p, k):
                return gidx[p].at[c * IB + k]

            def sref(p, k):
                return sidx[p].at[c * IB + k]
        else:
            def start_idx(b, p):
                blk = (c * NSUB + s) * NB + b
                pltpu.async_copy(gidx_hbm.at[blk], gidx[p], sem_i[p])
                pltpu.async_copy(sidx_hbm.at[blk], sidx[p], sem_i[p])

            def wait_idx(p):
                pltpu.make_async_copy(gidx_hbm.at[0], gidx[p], sem_i[p]).wait()
                pltpu.make_async_copy(sidx_hbm.at[0], sidx[p], sem_i[p]).wait()

            def gref(p, k):
                return gidx[p].at[k]

            def sref(p, k):
                return sidx[p].at[k]

        def start_gather(idx_ref, buf):
            pltpu.async_copy(table_hbm.at[idx_ref], buf, sem_g)

        def wait_gather(p):
            pltpu.make_async_copy(table_hbm.at[gref(0, 0)], rows[p], sem_g).wait()

        def start_scatter(buf, idx_ref):
            pltpu.async_copy(buf, acc_sh.at[idx_ref], sem_s, add=True)

        def wait_scatter(p):
            pltpu.make_async_copy(rows[p], acc_sh.at[sref(0, 0)], sem_s).wait()

        start_idx(0, 0)
        wait_idx(0)
        start_gather(gref(0, 0), rows0)

        def sb(g, _):
            not_last = g < NSB - 1
            for k in range(2 * IB):
                p = k % 2
                cur, nxt = rows[p], rows[1 - p]
                if k == 0:
                    @pl.when(g > 0)
                    def _():
                        wait_scatter(1 - p)
                else:
                    wait_scatter(1 - p)
                if k == 1:
                    start_idx(2 * g + 1, 1)
                if k == IB:
                    @pl.when(not_last)
                    def _():
                        start_idx(2 * (g + 1), 0)
                if k == IB - 1:
                    wait_idx(1)
                if k < IB - 1:
                    start_gather(gref(0, k + 1), nxt)
                elif k < 2 * IB - 1:
                    start_gather(gref(1, k + 1 - IB), nxt)
                else:
                    @pl.when(not_last)
                    def _():
                        wait_idx(0)
                        start_gather(gref(0, 0), nxt)
                wait_gather(p)
                if k < IB:
                    start_scatter(cur, sref(0, k))
                else:
                    start_scatter(cur, sref(1, k - IB))
            return 0
        lax.fori_loop(0, NSB, sb, 0)
        wait_scatter(1)
        plsc.subcore_barrier()

        i16 = lax.broadcasted_iota(jnp.int32, (16,), 0)
        nd = D // 16

        def outcp(t, _):
            base = s * rps + t * OC
            obase = c * acc_rows + base
            if mode == "c":
                pltpu.sync_copy(acc_sh.at[pl.ds(base, OC)],
                                out_hbm.at[pl.ds(obase, OC)])
                return 0
            pltpu.sync_copy(acc_sh.at[pl.ds(base, OC)], rows0.at[pl.ds(0, OC)])
            if mode == "a1":
                pltpu.sync_copy(sval_hbm.at[pl.ds(base, OC)], sval_v)
            elif mode == "a2":
                pltpu.sync_copy(sval_hbm.at[pl.ds(obase, OC)], sval_v)
            if mode == "c2":
                @pl.when(base < N)
                def _():
                    pltpu.sync_copy(sval_hbm.at[pl.ds(c * N + base, OC)],
                                    rows1.at[pl.ds(0, OC)])
                    pltpu.sync_copy(sval2_hbm.at[pl.ds(c * N + base, OC)], sval_v)
            for q in range(OC // 16):
                rb = q * 16
                if mode == "a1":
                    homo16 = sval_v[pl.ds(rb, 16)]
                    cnt16 = plsc.load_gather(
                        rows0, [rb + i16, jnp.full((16,), NHID, jnp.int32)])
                    scale16 = homo16 / jnp.maximum(cnt16, 1.0)
                    g_v[pl.ds(rb, 16)] = scale16
                else:
                    scale16 = sval_v[pl.ds(rb, 16)]
                for r in range(16):
                    row = rb + r
                    sc16 = _bc(scale16, r)
                    nfeat_regs = (NHID // 16) if mode == "a1" else nd
                    for k2 in range(nfeat_regs):
                        if mode == "c2":
                            rows0[row, pl.ds(k2 * 16, 16)] = (
                                rows1[row, pl.ds(k2 * 16, 16)]
                                + rows0[row, pl.ds(k2 * 16, 16)] * sc16)
                        else:
                            rows0[row, pl.ds(k2 * 16, 16)] = (
                                rows0[row, pl.ds(k2 * 16, 16)] * sc16)
                    if mode == "a1":
                        h16 = _bc(homo16, r)
                        rows0[row, pl.ds(NHID, 16)] = jnp.where(
                            i16 == 0, h16,
                            jnp.where(i16 == 1, jnp.float32(1.0), jnp.float32(0.0)))
            pltpu.sync_copy(rows0.at[pl.ds(0, OC)], out_hbm.at[pl.ds(obase, OC)])
            if mode == "a1":
                pltpu.sync_copy(g_v, gout_hbm.at[pl.ds(obase, OC)])
            return 0
        lax.fori_loop(0, rps // OC, outcp, 0)

    kern_name = name

    cp = pltpu.CompilerParams(use_tc_tiling_on_sc=False,
                              needs_layout_passes=False)
    if mode == "a1":
        @functools.partial(pl.kernel, out_type=out_type, mesh=mesh,
                           scratch_types=scratch, name=kern_name,
                           compiler_params=cp)
        def sc_stage(gIA, gIB, sIA, sIB, table, homo_hbm, out, gout, *scr):
            sc_body((gIA, gIB, sIA, sIB, table, homo_hbm, None, out, gout) + scr)
    elif mode == "a2":
        @functools.partial(pl.kernel, out_type=out_type, mesh=mesh,
                           scratch_types=scratch, name=kern_name,
                           compiler_params=cp)
        def sc_stage(gIA, gIB, sIA, sIB, table, g_hbm, out, *scr):
            sc_body((gIA, gIB, sIA, sIB, table, g_hbm, None, out, None) + scr)
    elif mode == "c2":
        @functools.partial(pl.kernel, out_type=out_type, mesh=mesh,
                           scratch_types=scratch, name=kern_name,
                           compiler_params=cp)
        def sc_stage(gI, sI, table, y2_hbm, hv_hbm, out, *scr):
            sc_body((gI, None, sI, None, table, y2_hbm, hv_hbm, out, None) + scr)
    else:
        @functools.partial(pl.kernel, out_type=out_type, mesh=mesh,
                           scratch_types=scratch, name=kern_name,
                           compiler_params=cp)
        def sc_stage(gI, sI, table, out, *scr):
            sc_body((gI, None, sI, None, table, None, None, out, None) + scr)

    return sc_stage


_sc_a1 = _make_sc_stage(D1, SPE, "sc_edge_agg1", "a1", split_idx=True)
_sc_c1 = _make_sc_stage(D1, NP, "sc_vert_agg1", "c")
_sc_a2 = _make_sc_stage(D2, SPE, "sc_edge_agg2", "a2", split_idx=True)
_sc_c2 = _make_sc_stage(D2, NP, "sc_vert_agg2", "c2")

BLK = 1000
EBLK = 512


def _mm1_kernel(x_ref, w_ref, o_ref):
    y = jnp.dot(x_ref[...], w_ref[...], preferred_element_type=jnp.float32)
    col = lax.broadcasted_iota(jnp.int32, (BLK, 16), 1)
    aug = jnp.where(col == 0, 1.0, 0.0).astype(jnp.float32)
    o_ref[...] = jnp.concatenate([y, aug], axis=1)


def _d1mm2_kernel(y1a_ref, svp_ref, wout_ref, z_ref, y2_ref, hv_ref):
    y1 = y1a_ref[:, :NHID]
    sv = svp_ref[0]
    att = sv[:, NHID:NHID + 1]
    cnt = sv[:, NHID + 1:NHID + 2]
    hv = 1.0 / (jnp.maximum(att, 1e-30) * jnp.clip(cnt, 1.0, None))
    x1 = jnp.maximum(y1 + sv[:, :NHID] * hv, 0.0)
    z_ref[0] = x1
    y2_ref[0] = jnp.dot(x1, wout_ref[...], preferred_element_type=jnp.float32)
    hv_ref[0] = hv


def kernel(X, W1, Wout, homo, H1_V, H1_E, H2_V, H2_E):
    f32 = jnp.float32

    Y1a = pl.pallas_call(
        _mm1_kernel, grid=(N // BLK,),
        in_specs=[pl.BlockSpec((BLK, NFEAT), lambda i: (i, 0)),
                  pl.BlockSpec((NFEAT, NHID), lambda i: (0, 0))],
        out_specs=pl.BlockSpec((BLK, D1), lambda i: (i, 0)),
        out_shape=jax.ShapeDtypeStruct((N, D1), f32),
    )(X, W1)

    def rs(a):
        return a.reshape(NSUB * NB, IB, CH)

    V1r, V2r = rs(H1_V), rs(H2_V)
    E1r, E2r = rs(H1_E), rs(H2_E)
    V2Nr = rs(H2_V + N)

    def cat(a, b):
        return jnp.concatenate([a, b])

    V_plain = cat(V1r, V2r)
    E_off = cat(E1r, rs(H2_E + SPE))

    homo_p = jnp.pad(homo, (0, SPE - NE))

    Xe1a, gvec = _sc_a1(V1r, V2r, E1r, E2r, Y1a, homo_p)

    Sv1 = _sc_c1(E_off, V_plain, Xe1a)
    Sv1v = Sv1.reshape(2, NP, D1)

    Z, Y2, hv = pl.pallas_call(
        _d1mm2_kernel, grid=(2, N // BLK),
        in_specs=[pl.BlockSpec((BLK, D1), lambda v, i: (i, 0)),
                  pl.BlockSpec((1, BLK, D1), lambda v, i: (v, i, 0)),
                  pl.BlockSpec((NHID, NCLASS), lambda v, i: (0, 0))],
        out_specs=[pl.BlockSpec((1, BLK, NHID), lambda v, i: (v, i, 0)),
                   pl.BlockSpec((1, BLK, NCLASS), lambda v, i: (v, i, 0)),
                   pl.BlockSpec((1, BLK, 1), lambda v, i: (v, i, 0))],
        out_shape=[jax.ShapeDtypeStruct((2, N, NHID), f32),
                   jax.ShapeDtypeStruct((2, N, NCLASS), f32),
                   jax.ShapeDtypeStruct((2, N, 1), f32)],
    )(Y1a, Sv1v, Wout)

    Xe2 = _sc_a2(V1r, V2Nr, E1r, E2r, Y2.reshape(2 * N, D2), gvec)

    Xo = _sc_c2(E_off, V_plain, Xe2,
                Y2.reshape(2 * N, D2), hv.reshape(2 * N))
    Xov = Xo.reshape(2, NP, D2)

    return (Z[0], Z[1], Xov[0, :N], Xov[1, :N])


# speedup vs baseline: 23.4112x; 1.0793x over previous
import functools

import jax
import jax.numpy as jnp
from jax import lax
from jax.experimental import pallas as pl
from jax.experimental.pallas import tpu as pltpu
from jax.experimental.pallas import tpu_sc as plsc

N = 10000
NE = 5000
INC = 320000
NFEAT = 128
NHID = 128
NCLASS = 64

D1 = NHID + 16
D2 = NCLASS
SPE = 5120
NP = 10240
NSUB = 16
PER_SUB = INC // NSUB
CH = 125
NCHUNK = PER_SUB // CH
IB = 5
NB = NCHUNK // IB
NSB = NB // 2
OC = 80


def _bc(v, r):
    return v.at[jnp.full((16,), r, jnp.int32)].get(mode="promise_in_bounds")


def _make_sc_stage(D, acc_rows, name, mode, split_idx=False, ib=IB, nbuf=2):
    rps = acc_rows // NSUB
    mesh = plsc.VectorSubcoreMesh(core_axis_name="c", subcore_axis_name="s")
    nb = NCHUNK // ib
    nsb = nb // 2
    IR = 2 * ib if split_idx else ib

    out_type = jax.ShapeDtypeStruct((2 * acc_rows, D), jnp.float32)
    if mode == "a1":
        out_type = [out_type, jax.ShapeDtypeStruct((2 * acc_rows,), jnp.float32)]
    scratch = [
        pltpu.VMEM((IR, CH), jnp.int32),
        pltpu.VMEM((IR, CH), jnp.int32),
        pltpu.VMEM((IR, CH), jnp.int32),
        pltpu.VMEM((IR, CH), jnp.int32),
    ] + [pltpu.VMEM((CH, D), jnp.float32)] * nbuf + [
        pltpu.VMEM((OC,), jnp.float32),
        pltpu.VMEM((OC,), jnp.float32),
        pltpu.VMEM_SHARED((acc_rows, D), jnp.float32),
        pltpu.SemaphoreType.DMA,
        pltpu.SemaphoreType.DMA,
        pltpu.SemaphoreType.DMA,
        pltpu.SemaphoreType.DMA,
    ]

    def sc_body(refs):
        (gidx_hbm, gidx_hbmB, sidx_hbm, sidx_hbmB, table_hbm, sval_hbm,
         sval2_hbm, out_hbm, gout_hbm,
         gidx0, gidx1, sidx0, sidx1) = refs[:13]
        rows = refs[13:13 + nbuf]
        (sval_v, g_v, acc_sh, sem_g, sem_s, sem_i0, sem_i1) = refs[13 + nbuf:]
        rows0, rows1 = rows[0], rows[1]
        c = lax.axis_index("c")
        s = lax.axis_index("s")
        gidx = (gidx0, gidx1)
        sidx = (sidx0, sidx1)
        sem_i = (sem_i0, sem_i1)

        nz = D // 16
        def zrow(i, _):
            rows0[i // nz, pl.ds((i % nz) * 16, 16)] = jnp.zeros((16,), jnp.float32)
            return 0
        lax.fori_loop(0, OC * nz, zrow, 0)

        def zacc(t, _):
            pltpu.sync_copy(rows0.at[pl.ds(0, OC)],
                            acc_sh.at[pl.ds(s * rps + t * OC, OC)])
            return 0
        lax.fori_loop(0, rps // OC, zacc, 0)
        plsc.subcore_barrier()

        if split_idx:
            def start_idx(b, p):
                blk = s * nb + b
                pltpu.async_copy(gidx_hbm.at[blk], gidx[p].at[pl.ds(0, ib)], sem_i[p])
                pltpu.async_copy(gidx_hbmB.at[blk], gidx[p].at[pl.ds(ib, ib)], sem_i[p])
                pltpu.async_copy(sidx_hbm.at[blk], sidx[p].at[pl.ds(0, ib)], sem_i[p])
                pltpu.async_copy(sidx_hbmB.at[blk], sidx[p].at[pl.ds(ib, ib)], sem_i[p])

            def wait_idx(p):
                for _ in range(2):
                    pltpu.make_async_copy(gidx_hbm.at[0], gidx[p].at[pl.ds(0, ib)], sem_i[p]).wait()
                    pltpu.make_async_copy(sidx_hbm.at[0], sidx[p].at[pl.ds(0, ib)], sem_i[p]).wait()

            def gref(p, k):
                return gidx[p].at[c * ib + k]

            def sref(p, k):
                return sidx[p].at[c * ib + k]
        else:
            def start_idx(b, p):
                blk = (c * NSUB + s) * nb + b
                pltpu.async_copy(gidx_hbm.at[blk], gidx[p], sem_i[p])
                pltpu.async_copy(sidx_hbm.at[blk], sidx[p], sem_i[p])

            def wait_idx(p):
                pltpu.make_async_copy(gidx_hbm.at[0], gidx[p], sem_i[p]).wait()
                pltpu.make_async_copy(sidx_hbm.at[0], sidx[p], sem_i[p]).wait()

            def gref(p, k):
                return gidx[p].at[k]

            def sref(p, k):
                return sidx[p].at[k]

        def start_gather(idx_ref, buf):
            pltpu.async_copy(table_hbm.at[idx_ref], buf, sem_g)

        def wait_gather(p):
            pltpu.make_async_copy(table_hbm.at[gref(0, 0)], rows[p], sem_g).wait()

        def start_scatter(buf, idx_ref):
            pltpu.async_copy(buf, acc_sh.at[idx_ref], sem_s, add=True)

        def wait_scatter(p):
            pltpu.make_async_copy(rows[p], acc_sh.at[sref(0, 0)], sem_s).wait()

        start_idx(0, 0)
        wait_idx(0)
        start_gather(gref(0, 0), rows0)

        if nbuf == 2:
            def sb(g, _):
                not_last = g < nsb - 1
                for k in range(2 * ib):
                    p = k % 2
                    cur, nxt = rows[p], rows[1 - p]
                    if k == 0:
                        @pl.when(g > 0)
                        def _():
                            wait_scatter(1 - p)
                    else:
                        wait_scatter(1 - p)
                    if k == 1:
                        start_idx(2 * g + 1, 1)
                    if k == ib:
                        @pl.when(not_last)
                        def _():
                            start_idx(2 * (g + 1), 0)
                    if k == ib - 1:
                        wait_idx(1)
                    if k < ib - 1:
                        start_gather(gref(0, k + 1), nxt)
                    elif k < 2 * ib - 1:
                        start_gather(gref(1, k + 1 - ib), nxt)
                    else:
                        @pl.when(not_last)
                        def _():
                            wait_idx(0)
                            start_gather(gref(0, 0), nxt)
                    wait_gather(p)
                    if k < ib:
                        start_scatter(cur, sref(0, k))
                    else:
                        start_scatter(cur, sref(1, k - ib))
                return 0
            lax.fori_loop(0, nsb, sb, 0)
            wait_scatter(1)
        else:
            start_gather(gref(0, 1), rows[1])

            def sb(g, _):
                not_last = g < nsb - 1
                for k in range(2 * ib):
                    if k <= 1:
                        @pl.when(g > 0)
                        def _():
                            wait_scatter(0)
                    else:
                        wait_scatter(0)
                    if k == 2:
                        start_idx(2 * g + 1, 1)
                    if k == ib + 1:
                        @pl.when(not_last)
                        def _():
                            start_idx(2 * (g + 1), 0)
                    if k == ib - 2:
                        wait_idx(1)
                    tgt = rows[(k + 2) % 4]
                    if k <= ib - 3:
                        start_gather(gref(0, k + 2), tgt)
                    elif k <= 2 * ib - 3:
                        start_gather(gref(1, k + 2 - ib), tgt)
                    else:
                        @pl.when(not_last)
                        def _():
                            if k == 2 * ib - 2:
                                wait_idx(0)
                            start_gather(gref(0, k + 2 - 2 * ib), tgt)
                    wait_gather(k % 4)
                    if k < ib:
                        start_scatter(rows[k % 4], sref(0, k))
                    else:
                        start_scatter(rows[k % 4], sref(1, k - ib))
                return 0
            lax.fori_loop(0, nsb, sb, 0)
            wait_scatter(0)
            wait_scatter(0)
        plsc.subcore_barrier()

        i16 = lax.broadcasted_iota(jnp.int32, (16,), 0)
        nd = D // 16

        def outcp(t, _):
            base = s * rps + t * OC
            obase = c * acc_rows + base
            if mode == "c":
                pltpu.sync_copy(acc_sh.at[pl.ds(base, OC)],
                                out_hbm.at[pl.ds(obase, OC)])
                return 0
            pltpu.sync_copy(acc_sh.at[pl.ds(base, OC)], rows0.at[pl.ds(0, OC)])
            if mode == "a1":
                pltpu.sync_copy(sval_hbm.at[pl.ds(base, OC)], sval_v)
            elif mode == "a2":
                pltpu.sync_copy(sval_hbm.at[pl.ds(obase, OC)], sval_v)
            if mode == "c2":
                @pl.when(base < N)
                def _():
                    pltpu.sync_copy(sval_hbm.at[pl.ds(c * N + base, OC)],
                                    rows1.at[pl.ds(0, OC)])
                    pltpu.sync_copy(sval2_hbm.at[pl.ds(c * N + base, OC)], sval_v)
            for q in range(OC // 16):
                rb = q * 16
                if mode == "a1":
                    homo16 = sval_v[pl.ds(rb, 16)]
                    cnt16 = plsc.load_gather(
                        rows0, [rb + i16, jnp.full((16,), NHID, jnp.int32)])
                    scale16 = homo16 / jnp.maximum(cnt16, 1.0)
                    g_v[pl.ds(rb, 16)] = scale16
                else:
                    scale16 = sval_v[pl.ds(rb, 16)]
                for r in range(16):
                    row = rb + r
                    sc16 = _bc(scale16, r)
                    nfeat_regs = (NHID // 16) if mode == "a1" else nd
                    for k2 in range(nfeat_regs):
                        if mode == "c2":
                            rows0[row, pl.ds(k2 * 16, 16)] = (
                                rows1[row, pl.ds(k2 * 16, 16)]
                                + rows0[row, pl.ds(k2 * 16, 16)] * sc16)
                        else:
                            rows0[row, pl.ds(k2 * 16, 16)] = (
                                rows0[row, pl.ds(k2 * 16, 16)] * sc16)
                    if mode == "a1":
                        h16 = _bc(homo16, r)
                        rows0[row, pl.ds(NHID, 16)] = jnp.where(
                            i16 == 0, h16,
                            jnp.where(i16 == 1, jnp.float32(1.0), jnp.float32(0.0)))
            pltpu.sync_copy(rows0.at[pl.ds(0, OC)], out_hbm.at[pl.ds(obase, OC)])
            if mode == "a1":
                pltpu.sync_copy(g_v, gout_hbm.at[pl.ds(obase, OC)])
            return 0
        lax.fori_loop(0, rps // OC, outcp, 0)

    kern_name = name

    cp = pltpu.CompilerParams(use_tc_tiling_on_sc=False,
                              needs_layout_passes=False)
    if mode == "a1":
        @functools.partial(pl.kernel, out_type=out_type, mesh=mesh,
                           scratch_types=scratch, name=kern_name,
                           compiler_params=cp)
        def sc_stage(gIA, gIB, sIA, sIB, table, homo_hbm, out, gout, *scr):
            sc_body((gIA, gIB, sIA, sIB, table, homo_hbm, None, out, gout) + scr)
    elif mode == "a2":
        @functools.partial(pl.kernel, out_type=out_type, mesh=mesh,
                           scratch_types=scratch, name=kern_name,
                           compiler_params=cp)
        def sc_stage(gIA, gIB, sIA, sIB, table, g_hbm, out, *scr):
            sc_body((gIA, gIB, sIA, sIB, table, g_hbm, None, out, None) + scr)
    elif mode == "c2":
        @functools.partial(pl.kernel, out_type=out_type, mesh=mesh,
                           scratch_types=scratch, name=kern_name,
                           compiler_params=cp)
        def sc_stage(gI, sI, table, y2_hbm, hv_hbm, out, *scr):
            sc_body((gI, None, sI, None, table, y2_hbm, hv_hbm, out, None) + scr)
    else:
        @functools.partial(pl.kernel, out_type=out_type, mesh=mesh,
                           scratch_types=scratch, name=kern_name,
                           compiler_params=cp)
        def sc_stage(gI, sI, table, out, *scr):
            sc_body((gI, None, sI, None, table, None, None, out, None) + scr)

    return sc_stage


_sc_a1 = _make_sc_stage(D1, SPE, "sc_edge_agg1", "a1", split_idx=True)
_sc_c1 = _make_sc_stage(D1, NP, "sc_vert_agg1", "c")
IB2 = 2 * IB
_sc_a2 = _make_sc_stage(D2, SPE, "sc_edge_agg2", "a2", split_idx=True,
                        ib=IB2, nbuf=4)
_sc_c2 = _make_sc_stage(D2, NP, "sc_vert_agg2", "c2", ib=IB2, nbuf=4)

BLK = 1000
EBLK = 512


def _mm1_kernel(x_ref, w_ref, o_ref):
    y = jnp.dot(x_ref[...], w_ref[...], preferred_element_type=jnp.float32)
    col = lax.broadcasted_iota(jnp.int32, (BLK, 16), 1)
    aug = jnp.where(col == 0, 1.0, 0.0).astype(jnp.float32)
    o_ref[...] = jnp.concatenate([y, aug], axis=1)


def _d1mm2_kernel(y1a_ref, svp_ref, wout_ref, z_ref, y2_ref, hv_ref):
    y1 = y1a_ref[:, :NHID]
    sv = svp_ref[0]
    att = sv[:, NHID:NHID + 1]
    cnt = sv[:, NHID + 1:NHID + 2]
    hv = 1.0 / (jnp.maximum(att, 1e-30) * jnp.clip(cnt, 1.0, None))
    x1 = jnp.maximum(y1 + sv[:, :NHID] * hv, 0.0)
    z_ref[0] = x1
    y2_ref[0] = jnp.dot(x1, wout_ref[...], preferred_element_type=jnp.float32)
    hv_ref[0] = hv


def kernel(X, W1, Wout, homo, H1_V, H1_E, H2_V, H2_E):
    f32 = jnp.float32

    Y1a = pl.pallas_call(
        _mm1_kernel, grid=(N // BLK,),
        in_specs=[pl.BlockSpec((BLK, NFEAT), lambda i: (i, 0)),
                  pl.BlockSpec((NFEAT, NHID), lambda i: (0, 0))],
        out_specs=pl.BlockSpec((BLK, D1), lambda i: (i, 0)),
        out_shape=jax.ShapeDtypeStruct((N, D1), f32),
    )(X, W1)

    def rs(a):
        return a.reshape(NSUB * NB, IB, CH)

    V1r, V2r = rs(H1_V), rs(H2_V)
    E1r, E2r = rs(H1_E), rs(H2_E)
    V2Nr = rs(H2_V + N)

    def cat(a, b):
        return jnp.concatenate([a, b])

    V_plain = cat(V1r, V2r)
    E_off = cat(E1r, rs(H2_E + SPE))

    homo_p = jnp.pad(homo, (0, SPE - NE))

    Xe1a, gvec = _sc_a1(V1r, V2r, E1r, E2r, Y1a, homo_p)

    Sv1 = _sc_c1(E_off, V_plain, Xe1a)
    Sv1v = Sv1.reshape(2, NP, D1)

    Z, Y2, hv = pl.pallas_call(
        _d1mm2_kernel, grid=(2, N // BLK),
        in_specs=[pl.BlockSpec((BLK, D1), lambda v, i: (i, 0)),
                  pl.BlockSpec((1, BLK, D1), lambda v, i: (v, i, 0)),
                  pl.BlockSpec((NHID, NCLASS), lambda v, i: (0, 0))],
        out_specs=[pl.BlockSpec((1, BLK, NHID), lambda v, i: (v, i, 0)),
                   pl.BlockSpec((1, BLK, NCLASS), lambda v, i: (v, i, 0)),
                   pl.BlockSpec((1, BLK, 1), lambda v, i: (v, i, 0))],
        out_shape=[jax.ShapeDtypeStruct((2, N, NHID), f32),
                   jax.ShapeDtypeStruct((2, N, NCLASS), f32),
                   jax.ShapeDtypeStruct((2, N, 1), f32)],
    )(Y1a, Sv1v, Wout)

    def rb2(a):
        return a.reshape(a.shape[0] // 2, IB2, CH)

    Xe2 = _sc_a2(rb2(V1r), rb2(V2Nr), rb2(E1r), rb2(E2r),
                 Y2.reshape(2 * N, D2), gvec)

    Xo = _sc_c2(rb2(E_off), rb2(V_plain), Xe2,
                Y2.reshape(2 * N, D2), hv.reshape(2 * N))
    Xov = Xo.reshape(2, NP, D2)

    return (Z[0], Z[1], Xov[0, :N], Xov[1, :N])


# speedup vs baseline: 24.7835x; 1.0586x over previous
import functools

import jax
import jax.numpy as jnp
from jax import lax
from jax.experimental import pallas as pl
from jax.experimental.pallas import tpu as pltpu
from jax.experimental.pallas import tpu_sc as plsc

N = 10000
NE = 5000
INC = 320000
NFEAT = 128
NHID = 128
NCLASS = 64

D1 = NHID + 16
D2 = NCLASS
SPE = 5120
NP = 10240
NSUB = 16
PER_SUB = INC // NSUB
CH = 125
NCHUNK = PER_SUB // CH
IB = 5
NB = NCHUNK // IB
NSB = NB // 2
OC = 80


def _bc(v, r):
    return v.at[jnp.full((16,), r, jnp.int32)].get(mode="promise_in_bounds")


def _make_sc_stage(D, acc_rows, name, mode, split_idx=False, ib=IB, nbuf=2):
    rps = acc_rows // NSUB
    mesh = plsc.VectorSubcoreMesh(core_axis_name="c", subcore_axis_name="s")
    nb = NCHUNK // ib
    nsb = nb // 2
    IR = 2 * ib if split_idx else ib

    out_type = jax.ShapeDtypeStruct((2 * acc_rows, D), jnp.float32)
    if mode == "a1":
        out_type = [out_type, jax.ShapeDtypeStruct((2 * acc_rows,), jnp.float32)]
    scratch = [
        pltpu.VMEM((IR, CH), jnp.int32),
        pltpu.VMEM((IR, CH), jnp.int32),
        pltpu.VMEM((IR, CH), jnp.int32),
        pltpu.VMEM((IR, CH), jnp.int32),
    ] + [pltpu.VMEM((CH, D), jnp.float32)] * nbuf + [
        pltpu.VMEM((OC,), jnp.float32),
        pltpu.VMEM((OC,), jnp.float32),
        pltpu.VMEM_SHARED((acc_rows, D), jnp.float32),
        pltpu.SemaphoreType.DMA,
        pltpu.SemaphoreType.DMA,
        pltpu.SemaphoreType.DMA,
        pltpu.SemaphoreType.DMA,
    ]

    def sc_body(refs):
        (gidx_hbm, gidx_hbmB, sidx_hbm, sidx_hbmB, table_hbm, sval_hbm,
         sval2_hbm, out_hbm, gout_hbm,
         gidx0, gidx1, sidx0, sidx1) = refs[:13]
        rows = refs[13:13 + nbuf]
        (sval_v, g_v, acc_sh, sem_g, sem_s, sem_i0, sem_i1) = refs[13 + nbuf:]
        rows0, rows1 = rows[0], rows[1]
        c = lax.axis_index("c")
        s = lax.axis_index("s")
        gidx = (gidx0, gidx1)
        sidx = (sidx0, sidx1)
        sem_i = (sem_i0, sem_i1)

        nz = D // 16
        def zrow(i, _):
            rows0[i // nz, pl.ds((i % nz) * 16, 16)] = jnp.zeros((16,), jnp.float32)
            return 0
        lax.fori_loop(0, OC * nz, zrow, 0)

        def zacc(t, _):
            pltpu.sync_copy(rows0.at[pl.ds(0, OC)],
                            acc_sh.at[pl.ds(s * rps + t * OC, OC)])
            return 0
        lax.fori_loop(0, rps // OC, zacc, 0)
        plsc.subcore_barrier()

        if split_idx:
            def start_idx(b, p):
                blk = s * nb + b
                pltpu.async_copy(gidx_hbm.at[blk], gidx[p].at[pl.ds(0, ib)], sem_i[p])
                pltpu.async_copy(gidx_hbmB.at[blk], gidx[p].at[pl.ds(ib, ib)], sem_i[p])
                pltpu.async_copy(sidx_hbm.at[blk], sidx[p].at[pl.ds(0, ib)], sem_i[p])
                pltpu.async_copy(sidx_hbmB.at[blk], sidx[p].at[pl.ds(ib, ib)], sem_i[p])

            def wait_idx(p):
                for _ in range(2):
                    pltpu.make_async_copy(gidx_hbm.at[0], gidx[p].at[pl.ds(0, ib)], sem_i[p]).wait()
                    pltpu.make_async_copy(sidx_hbm.at[0], sidx[p].at[pl.ds(0, ib)], sem_i[p]).wait()

            def gref(p, k):
                return gidx[p].at[c * ib + k]

            def sref(p, k):
                return sidx[p].at[c * ib + k]
        else:
            def start_idx(b, p):
                blk = (c * NSUB + s) * nb + b
                pltpu.async_copy(gidx_hbm.at[blk], gidx[p], sem_i[p])
                pltpu.async_copy(sidx_hbm.at[blk], sidx[p], sem_i[p])

            def wait_idx(p):
                pltpu.make_async_copy(gidx_hbm.at[0], gidx[p], sem_i[p]).wait()
                pltpu.make_async_copy(sidx_hbm.at[0], sidx[p], sem_i[p]).wait()

            def gref(p, k):
                return gidx[p].at[k]

            def sref(p, k):
                return sidx[p].at[k]

        def start_gather(idx_ref, buf):
            pltpu.async_copy(table_hbm.at[idx_ref], buf, sem_g)

        def wait_gather(p):
            pltpu.make_async_copy(table_hbm.at[gref(0, 0)], rows[p], sem_g).wait()

        def start_scatter(buf, idx_ref):
            pltpu.async_copy(buf, acc_sh.at[idx_ref], sem_s, add=True)

        def wait_scatter(p):
            pltpu.make_async_copy(rows[p], acc_sh.at[sref(0, 0)], sem_s).wait()

        start_idx(0, 0)
        wait_idx(0)
        start_gather(gref(0, 0), rows0)

        if nbuf == 2:
            def sb(g, _):
                not_last = g < nsb - 1
                for k in range(2 * ib):
                    p = k % 2
                    cur, nxt = rows[p], rows[1 - p]
                    if k == 0:
                        @pl.when(g > 0)
                        def _():
                            wait_scatter(1 - p)
                    else:
                        wait_scatter(1 - p)
                    if k == 1:
                        start_idx(2 * g + 1, 1)
                    if k == ib:
                        @pl.when(not_last)
                        def _():
                            start_idx(2 * (g + 1), 0)
                    if k == ib - 1:
                        wait_idx(1)
                    if k < ib - 1:
                        start_gather(gref(0, k + 1), nxt)
                    elif k < 2 * ib - 1:
                        start_gather(gref(1, k + 1 - ib), nxt)
                    else:
                        @pl.when(not_last)
                        def _():
                            wait_idx(0)
                            start_gather(gref(0, 0), nxt)
                    wait_gather(p)
                    if k < ib:
                        start_scatter(cur, sref(0, k))
                    else:
                        start_scatter(cur, sref(1, k - ib))
                return 0
            lax.fori_loop(0, nsb, sb, 0)
            wait_scatter(1)
        else:
            start_gather(gref(0, 1), rows[1])

            def sb(g, _):
                not_last = g < nsb - 1
                for k in range(2 * ib):
                    if k <= 1:
                        @pl.when(g > 0)
                        def _():
                            wait_scatter(0)
                    else:
                        wait_scatter(0)
                    if k == 2:
                        start_idx(2 * g + 1, 1)
                    if k == ib + 1:
                        @pl.when(not_last)
                        def _():
                            start_idx(2 * (g + 1), 0)
                    if k == ib - 2:
                        wait_idx(1)
                    tgt = rows[(k + 2) % 4]
                    if k <= ib - 3:
                        start_gather(gref(0, k + 2), tgt)
                    elif k <= 2 * ib - 3:
                        start_gather(gref(1, k + 2 - ib), tgt)
                    else:
                        @pl.when(not_last)
                        def _():
                            if k == 2 * ib - 2:
                                wait_idx(0)
                            start_gather(gref(0, k + 2 - 2 * ib), tgt)
                    wait_gather(k % 4)
                    if k < ib:
                        start_scatter(rows[k % 4], sref(0, k))
                    else:
                        start_scatter(rows[k % 4], sref(1, k - ib))
                return 0
            lax.fori_loop(0, nsb, sb, 0)
            wait_scatter(0)
            wait_scatter(0)
        plsc.subcore_barrier()

        i16 = lax.broadcasted_iota(jnp.int32, (16,), 0)
        nd = D // 16

        def outcp(t, _):
            base = s * rps + t * OC
            obase = c * acc_rows + base
            if mode == "c":
                pltpu.sync_copy(acc_sh.at[pl.ds(base, OC)],
                                out_hbm.at[pl.ds(obase, OC)])
                return 0
            pltpu.sync_copy(acc_sh.at[pl.ds(base, OC)], rows0.at[pl.ds(0, OC)])
            if mode == "a1":
                pltpu.sync_copy(sval_hbm.at[pl.ds(base, OC)], sval_v)
            elif mode == "a2":
                pltpu.sync_copy(sval_hbm.at[pl.ds(obase, OC)], sval_v)
            if mode == "c2":
                @pl.when(base < N)
                def _():
                    pltpu.sync_copy(sval_hbm.at[pl.ds(c * N + base, OC)],
                                    rows1.at[pl.ds(0, OC)])
                    pltpu.sync_copy(sval2_hbm.at[pl.ds(c * N + base, OC)], sval_v)
            for q in range(OC // 16):
                rb = q * 16
                if mode == "a1":
                    homo16 = sval_v[pl.ds(rb, 16)]
                    cnt16 = plsc.load_gather(
                        rows0, [rb + i16, jnp.full((16,), NHID, jnp.int32)])
                    scale16 = homo16 / jnp.maximum(cnt16, 1.0)
                    g_v[pl.ds(rb, 16)] = scale16
                else:
                    scale16 = sval_v[pl.ds(rb, 16)]
                for r in range(16):
                    row = rb + r
                    sc16 = _bc(scale16, r)
                    nfeat_regs = (NHID // 16) if mode == "a1" else nd
                    for k2 in range(nfeat_regs):
                        if mode == "c2":
                            rows0[row, pl.ds(k2 * 16, 16)] = (
                                rows1[row, pl.ds(k2 * 16, 16)]
                                + rows0[row, pl.ds(k2 * 16, 16)] * sc16)
                        else:
                            rows0[row, pl.ds(k2 * 16, 16)] = (
                                rows0[row, pl.ds(k2 * 16, 16)] * sc16)
                    if mode == "a1":
                        h16 = _bc(homo16, r)
                        rows0[row, pl.ds(NHID, 16)] = jnp.where(
                            i16 == 0, h16,
                            jnp.where(i16 == 1, jnp.float32(1.0), jnp.float32(0.0)))
            pltpu.sync_copy(rows0.at[pl.ds(0, OC)], out_hbm.at[pl.ds(obase, OC)])
            if mode == "a1":
                pltpu.sync_copy(g_v, gout_hbm.at[pl.ds(obase, OC)])
            return 0
        lax.fori_loop(0, rps // OC, outcp, 0)

    kern_name = name

    cp = pltpu.CompilerParams(use_tc_tiling_on_sc=False,
                              needs_layout_passes=False)
    if mode == "a1":
        @functools.partial(pl.kernel, out_type=out_type, mesh=mesh,
                           scratch_types=scratch, name=kern_name,
                           compiler_params=cp)
        def sc_stage(gIA, gIB, sIA, sIB, table, homo_hbm, out, gout, *scr):
            sc_body((gIA, gIB, sIA, sIB, table, homo_hbm, None, out, gout) + scr)
    elif mode == "a2":
        @functools.partial(pl.kernel, out_type=out_type, mesh=mesh,
                           scratch_types=scratch, name=kern_name,
                           compiler_params=cp)
        def sc_stage(gIA, gIB, sIA, sIB, table, g_hbm, out, *scr):
            sc_body((gIA, gIB, sIA, sIB, table, g_hbm, None, out, None) + scr)
    elif mode == "c2":
        @functools.partial(pl.kernel, out_type=out_type, mesh=mesh,
                           scratch_types=scratch, name=kern_name,
                           compiler_params=cp)
        def sc_stage(gI, sI, table, y2_hbm, hv_hbm, out, *scr):
            sc_body((gI, None, sI, None, table, y2_hbm, hv_hbm, out, None) + scr)
    else:
        @functools.partial(pl.kernel, out_type=out_type, mesh=mesh,
                           scratch_types=scratch, name=kern_name,
                           compiler_params=cp)
        def sc_stage(gI, sI, table, out, *scr):
            sc_body((gI, None, sI, None, table, None, None, out, None) + scr)

    return sc_stage


IB2 = 2 * IB
_sc_a1 = _make_sc_stage(D1, SPE, "sc_edge_agg1", "a1", split_idx=True,
                        ib=IB2, nbuf=4)
_sc_c1 = _make_sc_stage(D1, NP, "sc_vert_agg1", "c")
_sc_a2 = _make_sc_stage(D2, SPE, "sc_edge_agg2", "a2", split_idx=True,
                        ib=IB2, nbuf=4)
_sc_c2 = _make_sc_stage(D2, NP, "sc_vert_agg2", "c2", ib=IB2, nbuf=4)

BLK = 1000
EBLK = 512


def _mm1_kernel(x_ref, w_ref, o_ref):
    y = jnp.dot(x_ref[...], w_ref[...], preferred_element_type=jnp.float32)
    col = lax.broadcasted_iota(jnp.int32, (BLK, 16), 1)
    aug = jnp.where(col == 0, 1.0, 0.0).astype(jnp.float32)
    o_ref[...] = jnp.concatenate([y, aug], axis=1)


def _d1mm2_kernel(y1a_ref, svp_ref, wout_ref, z_ref, y2_ref, hv_ref):
    y1 = y1a_ref[:, :NHID]
    sv = svp_ref[0]
    att = sv[:, NHID:NHID + 1]
    cnt = sv[:, NHID + 1:NHID + 2]
    hv = 1.0 / (jnp.maximum(att, 1e-30) * jnp.clip(cnt, 1.0, None))
    x1 = jnp.maximum(y1 + sv[:, :NHID] * hv, 0.0)
    z_ref[0] = x1
    y2_ref[0] = jnp.dot(x1, wout_ref[...], preferred_element_type=jnp.float32)
    hv_ref[0] = hv


def kernel(X, W1, Wout, homo, H1_V, H1_E, H2_V, H2_E):
    f32 = jnp.float32

    Y1a = pl.pallas_call(
        _mm1_kernel, grid=(N // BLK,),
        in_specs=[pl.BlockSpec((BLK, NFEAT), lambda i: (i, 0)),
                  pl.BlockSpec((NFEAT, NHID), lambda i: (0, 0))],
        out_specs=pl.BlockSpec((BLK, D1), lambda i: (i, 0)),
        out_shape=jax.ShapeDtypeStruct((N, D1), f32),
    )(X, W1)

    def rs(a):
        return a.reshape(NSUB * NB, IB, CH)

    V1r, V2r = rs(H1_V), rs(H2_V)
    E1r, E2r = rs(H1_E), rs(H2_E)
    V2Nr = rs(H2_V + N)

    def cat(a, b):
        return jnp.concatenate([a, b])

    V_plain = cat(V1r, V2r)
    E_off = cat(E1r, rs(H2_E + SPE))

    homo_p = jnp.pad(homo, (0, SPE - NE))

    def rb2(a):
        return a.reshape(a.shape[0] // 2, IB2, CH)

    Xe1a, gvec = _sc_a1(rb2(V1r), rb2(V2r), rb2(E1r), rb2(E2r),
                        Y1a, homo_p)

    Sv1 = _sc_c1(E_off, V_plain, Xe1a)
    Sv1v = Sv1.reshape(2, NP, D1)

    Z, Y2, hv = pl.pallas_call(
        _d1mm2_kernel, grid=(2, N // BLK),
        in_specs=[pl.BlockSpec((BLK, D1), lambda v, i: (i, 0)),
                  pl.BlockSpec((1, BLK, D1), lambda v, i: (v, i, 0)),
                  pl.BlockSpec((NHID, NCLASS), lambda v, i: (0, 0))],
        out_specs=[pl.BlockSpec((1, BLK, NHID), lambda v, i: (v, i, 0)),
                   pl.BlockSpec((1, BLK, NCLASS), lambda v, i: (v, i, 0)),
                   pl.BlockSpec((1, BLK, 1), lambda v, i: (v, i, 0))],
        out_shape=[jax.ShapeDtypeStruct((2, N, NHID), f32),
                   jax.ShapeDtypeStruct((2, N, NCLASS), f32),
                   jax.ShapeDtypeStruct((2, N, 1), f32)],
    )(Y1a, Sv1v, Wout)

    Xe2 = _sc_a2(rb2(V1r), rb2(V2Nr), rb2(E1r), rb2(E2r),
                 Y2.reshape(2 * N, D2), gvec)

    Xo = _sc_c2(rb2(E_off), rb2(V_plain), Xe2,
                Y2.reshape(2 * N, D2), hv.reshape(2 * N))
    Xov = Xo.reshape(2, NP, D2)

    return (Z[0], Z[1], Xov[0, :N], Xov[1, :N])
